# baseline (speedup 1.0000x reference)
import jax
import jax.numpy as jnp
from jax.experimental import pallas as pl
from jax.experimental.pallas import tpu as pltpu


def kernel(z, perm):
    B, C, H, W = z.shape
    D = H * W
    S = D // 128
    z2 = z.reshape(B, C, S, 128)

    def body(perm_ref, in_ref, out_ref):
        out_ref[...] = in_ref[...]

    out = pl.pallas_call(
        body,
        grid_spec=pltpu.PrefetchScalarGridSpec(
            num_scalar_prefetch=1,
            grid=(B, C),
            in_specs=[
                pl.BlockSpec(
                    (1, 1, S, 128), lambda b, c, perm_ref: (b, perm_ref[c], 0, 0)
                )
            ],
            out_specs=pl.BlockSpec((1, 1, S, 128), lambda b, c, perm_ref: (b, c, 0, 0)),
        ),
        out_shape=jax.ShapeDtypeStruct((B, C, S, 128), z.dtype),
    )(perm, z2)
    return out.reshape(B, C, H, W), jnp.zeros((), z.dtype)


# speedup vs baseline: 1.8878x; 1.8878x over previous
import functools

import jax
import jax.numpy as jnp
from jax import lax
from jax.experimental import pallas as pl
from jax.experimental.pallas import tpu as pltpu
from jax.experimental.pallas import tpu_sc as plsc


def _sc_permute(z2, rows2, *, R, D, NC, NS):
    NW = NC * NS
    RPW = R // NW

    mesh = plsc.VectorSubcoreMesh(core_axis_name="c", subcore_axis_name="s")

    @functools.partial(
        pl.kernel,
        mesh=mesh,
        out_type=jax.ShapeDtypeStruct((R, D), jnp.float32),
        scratch_types=[
            pltpu.VMEM((RPW, 1), jnp.int32),
            pltpu.VMEM((2, 1, D), jnp.float32),
            pltpu.SemaphoreType.DMA,
            pltpu.SemaphoreType.DMA,
            pltpu.SemaphoreType.DMA,
            pltpu.SemaphoreType.DMA,
        ],
    )
    def sc_copy(z_hbm, rows_hbm, out_hbm, idx_v, buf_v, g0, g1, s0, s1):
        gsem = (g0, g1)
        ssem = (s0, s1)
        wid = lax.axis_index("s") * NC + lax.axis_index("c")
        base = wid * RPW
        pltpu.sync_copy(rows_hbm.at[pl.ds(base, RPW)], idx_v)

        def scatter_start(r, b):
            pltpu.make_async_copy(
                buf_v.at[b], out_hbm.at[pl.ds(base + r, 1)], ssem[b]
            ).start()

        def scatter_wait(b):
            pltpu.make_async_copy(
                buf_v.at[b], out_hbm.at[pl.ds(0, 1)], ssem[b]
            ).wait()

        def body(p, _):
            for b in range(2):
                r = 2 * p + b

                @pl.when(p > 0)
                def _():
                    scatter_wait(b)

                pltpu.async_copy(
                    z_hbm.at[idx_v.at[r]], buf_v.at[b], gsem[b]
                ).wait()
                scatter_start(r, b)
            return 0

        lax.fori_loop(0, RPW // 2, body, 0)
        scatter_wait(0)
        scatter_wait(1)

    return sc_copy(z2, rows2)


def kernel(z, perm):
    B, C, H, W = z.shape
    D = H * W
    R = B * C
    z2 = z.reshape(R, D)
    rows = (
        jnp.arange(B, dtype=jnp.int32)[:, None] * C + perm[None, :].astype(jnp.int32)
    ).reshape(R, 1)
    info = plsc.get_sparse_core_info()
    out = _sc_permute(
        z2, rows, R=R, D=D, NC=info.num_cores, NS=info.num_subcores
    )
    return out.reshape(B, C, H, W), jnp.zeros((), z.dtype)
